# baseline (device time: 11875 ns/iter reference)
import jax
import jax.numpy as jnp
from jax import lax
from jax.experimental import pallas as pl
from jax.experimental.pallas import tpu as pltpu

M = 512
N_HALF = 512
M_HALF = M // 2


def kernel(x):
    def body(x_ref, out_ref, xrecv_ref, xsend_sem, xrecv_sem):
        my_x = lax.axis_index("x")
        my_y = lax.axis_index("y")
        partner_x = 1 - my_x

        barrier_sem = pltpu.get_barrier_semaphore()
        pl.semaphore_signal(
            barrier_sem, inc=1,
            device_id=(partner_x, my_y),
            device_id_type=pl.DeviceIdType.MESH,
        )
        pl.semaphore_wait(barrier_sem, 1)

        row0 = my_y * M_HALF
        keep_off = my_x * N_HALF
        send_off = partner_x * N_HALF

        rdma = pltpu.make_async_remote_copy(
            src_ref=x_ref.at[0, pl.ds(row0, M_HALF), pl.ds(send_off, N_HALF)],
            dst_ref=xrecv_ref,
            send_sem=xsend_sem,
            recv_sem=xrecv_sem,
            device_id=(partner_x, my_y),
            device_id_type=pl.DeviceIdType.MESH,
        )
        rdma.start()

        other_row0 = (1 - my_y) * M_HALF
        out_ref[pl.ds(other_row0, M_HALF), :] = x_ref[
            0, pl.ds(other_row0, M_HALF), pl.ds(keep_off, N_HALF)
        ]
        rdma.wait()
        out_ref[pl.ds(row0, M_HALF), :] = (
            x_ref[0, pl.ds(row0, M_HALF), pl.ds(keep_off, N_HALF)] + xrecv_ref[...]
        )

    return pl.pallas_call(
        body,
        out_shape=jax.ShapeDtypeStruct((M, N_HALF), jnp.float32),
        in_specs=[pl.BlockSpec(memory_space=pltpu.VMEM)],
        out_specs=pl.BlockSpec(memory_space=pltpu.VMEM),
        scratch_shapes=[
            pltpu.VMEM((M_HALF, N_HALF), jnp.float32),
            pltpu.SemaphoreType.DMA,
            pltpu.SemaphoreType.DMA,
        ],
        compiler_params=pltpu.CompilerParams(collective_id=0),
    )(x)
